# baseline (speedup 1.0000x reference)
import functools

import jax
import jax.numpy as jnp
from jax import lax
from jax.experimental import pallas as pl
from jax.experimental.pallas import tpu as pltpu
from jax.experimental.pallas import tpu_sc as plsc

EMBED = 64
CTX = 20
CTXP = 24
NSAMP = 64
NQ = EMBED // 16
TR_COLS = 16384
NBUF = 6
NOFF = CTXP
PR = CTXP + NSAMP


def _tr_body(u_ref, v_ref, out_ref):
    eye = jnp.eye(EMBED, dtype=jnp.float32)
    out_ref[:, 0:EMBED] = lax.dot_general(
        u_ref[...], eye, (((0,), (0,)), ((), ())),
        preferred_element_type=jnp.float32)
    out_ref[:, EMBED:2 * EMBED] = v_ref[...].T


def _combine_tables(ut, vt):
    v = ut.shape[1]
    grid = (v + TR_COLS - 1) // TR_COLS
    return pl.pallas_call(
        _tr_body,
        grid=(grid,),
        in_specs=[
            pl.BlockSpec((EMBED, TR_COLS), lambda i: (0, i)),
            pl.BlockSpec((EMBED, TR_COLS), lambda i: (0, i)),
        ],
        out_specs=pl.BlockSpec((TR_COLS, 2 * EMBED), lambda i: (i, 0)),
        out_shape=jax.ShapeDtypeStruct((v, 2 * EMBED), jnp.float32),
    )(ut, vt)


def _sc_gather_body(P_hbm, W_hbm, sim_hbm, neg_hbm,
                    pidx, wbuf, simloc, negloc, sems, *, per):
    c = lax.axis_index("c")
    s = lax.axis_index("s")
    wid = s * 2 + c
    base = wid * per

    pltpu.sync_copy(P_hbm.at[pl.ds(base * PR, per * PR)], pidx)

    def issue(buf, e):
        pltpu.make_async_copy(
            W_hbm.at[pidx.at[pl.ds(e * PR, PR)]], wbuf.at[buf],
            sems.at[buf]).start()

    def wait(buf):
        pltpu.make_async_copy(
            W_hbm.at[pidx.at[pl.ds(0, PR)]], wbuf.at[buf],
            sems.at[buf]).wait()

    zero = jnp.zeros((16,), jnp.float32)

    def compute(buf, e):
        wb = wbuf.at[buf]
        sim4 = [zero] * NQ
        for cc in range(CTX):
            for q in range(NQ):
                sim4[q] = sim4[q] + (wb[cc, pl.ds(q * 16, 16)] *
                                     wb[cc, pl.ds(EMBED + q * 16, 16)])
        for q in range(NQ):
            simloc[pl.ds(e * EMBED + q * 16, 16)] = sim4[q]

        su4 = [zero] * NQ
        for j in range(NSAMP):
            for q in range(NQ):
                su4[q] = su4[q] + wb[NOFF + j, pl.ds(q * 16, 16)]
        neg4 = [zero] * NQ
        for j in range(NSAMP):
            w = su4[j // 16][j % 16]
            for q in range(NQ):
                neg4[q] = neg4[q] + w * wb[NOFF + j, pl.ds(EMBED + q * 16, 16)]
        for q in range(NQ):
            negloc[pl.ds(e * EMBED + q * 16, 16)] = neg4[q]

    for b in range(NBUF):
        issue(b, b)

    @pl.loop(0, per, step=NBUF)
    def _pipe(e):
        for b in range(NBUF):
            wait(b)
            compute(b, e + b)
            issue(b, jnp.minimum(e + b + NBUF, per - 1))

    for b in range(NBUF):
        wait(b)

    pltpu.sync_copy(simloc, sim_hbm.at[pl.ds(base * EMBED, per * EMBED)])
    pltpu.sync_copy(negloc, neg_hbm.at[pl.ds(base * EMBED, per * EMBED)])


def _loss_body(sim_ref, neg_ref, out_ref, *, batch):
    x = sim_ref[...]
    y = -neg_ref[...]

    def log_sigmoid(t):
        return jnp.minimum(t, 0.0) - jnp.log1p(jnp.exp(-jnp.abs(t)))

    total = jnp.sum(log_sigmoid(x)) + jnp.sum(log_sigmoid(y))
    out_ref[0, 0] = -total / float(batch)


def kernel(X, N, neg_samples, batch_size, embed_u, embed_v):
    del N, batch_size
    B = X.shape[0]
    nw = 32
    per = B // nw

    W = _combine_tables(embed_u.T, embed_v.T)

    P = jnp.concatenate(
        [X, jnp.zeros((B, CTXP - CTX), X.dtype), neg_samples],
        axis=1).reshape(-1)

    mesh = plsc.VectorSubcoreMesh(core_axis_name="c", subcore_axis_name="s")
    sc = pl.kernel(
        functools.partial(_sc_gather_body, per=per),
        out_type=(
            jax.ShapeDtypeStruct((B * EMBED,), jnp.float32),
            jax.ShapeDtypeStruct((B * EMBED,), jnp.float32),
        ),
        mesh=mesh,
        scratch_types=(
            pltpu.VMEM((per * PR,), jnp.int32),
            pltpu.VMEM((NBUF, PR, 2 * EMBED), jnp.float32),
            pltpu.VMEM((per * EMBED,), jnp.float32),
            pltpu.VMEM((per * EMBED,), jnp.float32),
            pltpu.SemaphoreType.DMA((NBUF,)),
        ),
    )
    sim, neg = sc(P, W)

    loss = pl.pallas_call(
        functools.partial(_loss_body, batch=B),
        out_shape=jax.ShapeDtypeStruct((1, 1), jnp.float32),
        out_specs=pl.BlockSpec(memory_space=pltpu.SMEM),
    )(sim.reshape(B * EMBED // 128, 128), neg.reshape(B * EMBED // 128, 128))
    return loss[0, 0]


# speedup vs baseline: 1.0313x; 1.0313x over previous
import functools

import jax
import jax.numpy as jnp
from jax import lax
from jax.experimental import pallas as pl
from jax.experimental.pallas import tpu as pltpu
from jax.experimental.pallas import tpu_sc as plsc

EMBED = 64
CTX = 20
CTXP = 24
NSAMP = 64
NQ = EMBED // 16
TR_COLS = 16384
NBUF = 4
NOFF = CTXP
PR = CTXP + NSAMP


def _tr_body(u_ref, v_ref, out_ref):
    eye = jnp.eye(EMBED, dtype=jnp.float32)
    out_ref[:, 0:EMBED] = lax.dot_general(
        u_ref[...], eye, (((0,), (0,)), ((), ())),
        preferred_element_type=jnp.float32)
    out_ref[:, EMBED:2 * EMBED] = lax.dot_general(
        v_ref[...], eye, (((0,), (0,)), ((), ())),
        preferred_element_type=jnp.float32)


def _combine_tables(ut, vt):
    v = ut.shape[1]
    grid = (v + TR_COLS - 1) // TR_COLS
    return pl.pallas_call(
        _tr_body,
        grid=(grid,),
        in_specs=[
            pl.BlockSpec((EMBED, TR_COLS), lambda i: (0, i)),
            pl.BlockSpec((EMBED, TR_COLS), lambda i: (0, i)),
        ],
        out_specs=pl.BlockSpec((TR_COLS, 2 * EMBED), lambda i: (i, 0)),
        out_shape=jax.ShapeDtypeStruct((v, 2 * EMBED), jnp.float32),
    )(ut, vt)


def _sc_gather_body(P_hbm, W_hbm, sim_hbm, neg_hbm,
                    pidx, wbuf, simloc, negloc, sems, *, per):
    c = lax.axis_index("c")
    s = lax.axis_index("s")
    wid = s * 2 + c
    base = wid * per

    pltpu.sync_copy(P_hbm.at[pl.ds(base * PR, per * PR)], pidx)

    def issue(buf, e):
        pltpu.make_async_copy(
            W_hbm.at[pidx.at[pl.ds(e * PR, PR)]], wbuf.at[buf],
            sems.at[buf]).start()

    def wait(buf):
        pltpu.make_async_copy(
            W_hbm.at[pidx.at[pl.ds(0, PR)]], wbuf.at[buf],
            sems.at[buf]).wait()

    zero = jnp.zeros((16,), jnp.float32)

    def compute(buf, e):
        wb = wbuf.at[buf]
        sim4 = [zero] * NQ
        for cc in range(CTX):
            for q in range(NQ):
                sim4[q] = sim4[q] + (wb[cc, pl.ds(q * 16, 16)] *
                                     wb[cc, pl.ds(EMBED + q * 16, 16)])
        for q in range(NQ):
            simloc[pl.ds(e * EMBED + q * 16, 16)] = sim4[q]

        su4 = [zero] * NQ
        for j in range(NSAMP):
            for q in range(NQ):
                su4[q] = su4[q] + wb[NOFF + j, pl.ds(q * 16, 16)]
        neg4 = [zero] * NQ
        for j in range(NSAMP):
            w = su4[j // 16][j % 16]
            for q in range(NQ):
                neg4[q] = neg4[q] + w * wb[NOFF + j, pl.ds(EMBED + q * 16, 16)]
        for q in range(NQ):
            negloc[pl.ds(e * EMBED + q * 16, 16)] = neg4[q]

    for b in range(NBUF):
        issue(b, b)

    @pl.loop(0, per, step=NBUF)
    def _pipe(e):
        for b in range(NBUF):
            wait(b)
            compute(b, e + b)
            issue(b, jnp.minimum(e + b + NBUF, per - 1))

    for b in range(NBUF):
        wait(b)

    pltpu.sync_copy(simloc, sim_hbm.at[pl.ds(base * EMBED, per * EMBED)])
    pltpu.sync_copy(negloc, neg_hbm.at[pl.ds(base * EMBED, per * EMBED)])


def _loss_body(sim_ref, neg_ref, out_ref, *, batch):
    x = sim_ref[...]
    y = -neg_ref[...]

    def log_sigmoid(t):
        return jnp.minimum(t, 0.0) - jnp.log1p(jnp.exp(-jnp.abs(t)))

    total = jnp.sum(log_sigmoid(x)) + jnp.sum(log_sigmoid(y))
    out_ref[0, 0] = -total / float(batch)


def kernel(X, N, neg_samples, batch_size, embed_u, embed_v):
    del N, batch_size
    B = X.shape[0]
    nw = 32
    per = B // nw

    W = _combine_tables(embed_u.T, embed_v.T)

    P = jnp.concatenate(
        [X, jnp.zeros((B, CTXP - CTX), X.dtype), neg_samples],
        axis=1).reshape(-1)

    mesh = plsc.VectorSubcoreMesh(core_axis_name="c", subcore_axis_name="s")
    sc = pl.kernel(
        functools.partial(_sc_gather_body, per=per),
        out_type=(
            jax.ShapeDtypeStruct((B * EMBED,), jnp.float32),
            jax.ShapeDtypeStruct((B * EMBED,), jnp.float32),
        ),
        mesh=mesh,
        scratch_types=(
            pltpu.VMEM((per * PR,), jnp.int32),
            pltpu.VMEM((NBUF, PR, 2 * EMBED), jnp.float32),
            pltpu.VMEM((per * EMBED,), jnp.float32),
            pltpu.VMEM((per * EMBED,), jnp.float32),
            pltpu.SemaphoreType.DMA((NBUF,)),
        ),
    )
    sim, neg = sc(P, W)

    loss = pl.pallas_call(
        functools.partial(_loss_body, batch=B),
        out_shape=jax.ShapeDtypeStruct((1, 1), jnp.float32),
        out_specs=pl.BlockSpec(memory_space=pltpu.SMEM),
    )(sim.reshape(B * EMBED // 128, 128), neg.reshape(B * EMBED // 128, 128))
    return loss[0, 0]


# speedup vs baseline: 1.8486x; 1.7926x over previous
import functools

import jax
import jax.numpy as jnp
from jax import lax
from jax.experimental import pallas as pl
from jax.experimental.pallas import tpu as pltpu
from jax.experimental.pallas import tpu_sc as plsc

EMBED = 64
CTX = 20
CTXP = 24
NSAMP = 64
NQ = EMBED // 16
TR_COLS = 16384
NBUF = 4


def _tr_body(u_ref, v_ref, out_ref):
    eye = jnp.eye(EMBED, dtype=jnp.float32)
    out_ref[:, 0:EMBED] = lax.dot_general(
        u_ref[...], eye, (((0,), (0,)), ((), ())),
        preferred_element_type=jnp.float32)
    out_ref[:, EMBED:2 * EMBED] = lax.dot_general(
        v_ref[...], eye, (((0,), (0,)), ((), ())),
        preferred_element_type=jnp.float32)


def _combine_tables(ut, vt):
    v = ut.shape[1]
    grid = (v + TR_COLS - 1) // TR_COLS
    return pl.pallas_call(
        _tr_body,
        grid=(grid,),
        in_specs=[
            pl.BlockSpec((EMBED, TR_COLS), lambda i: (0, i)),
            pl.BlockSpec((EMBED, TR_COLS), lambda i: (0, i)),
        ],
        out_specs=pl.BlockSpec((TR_COLS, 2 * EMBED), lambda i: (i, 0)),
        out_shape=jax.ShapeDtypeStruct((v, 2 * EMBED), jnp.float32),
    )(ut, vt)


def _sc_gather_body(XP_hbm, NP_hbm, W_hbm, sim_hbm, neg_hbm,
                    xp, np_, wctx, wneg, simloc, negloc, sems, *, per):
    c = lax.axis_index("c")
    s = lax.axis_index("s")
    wid = s * 2 + c
    base = wid * per

    pltpu.sync_copy(XP_hbm.at[pl.ds(base * CTXP, per * CTXP)], xp)
    pltpu.sync_copy(NP_hbm.at[pl.ds(base * NSAMP, per * NSAMP)], np_)

    def issue(buf, e):
        pltpu.make_async_copy(
            W_hbm.at[xp.at[pl.ds(e * CTXP, CTX)]], wctx.at[buf],
            sems.at[buf]).start()
        pltpu.make_async_copy(
            W_hbm.at[np_.at[pl.ds(e * NSAMP, NSAMP)]], wneg.at[buf],
            sems.at[buf]).start()

    def wait(buf):
        pltpu.make_async_copy(
            W_hbm.at[xp.at[pl.ds(0, CTX)]], wctx.at[buf],
            sems.at[buf]).wait()
        pltpu.make_async_copy(
            W_hbm.at[np_.at[pl.ds(0, NSAMP)]], wneg.at[buf],
            sems.at[buf]).wait()

    zero = jnp.zeros((16,), jnp.float32)

    def compute(buf, e):
        wc, wn = wctx.at[buf], wneg.at[buf]
        sim4 = [zero] * NQ
        for cc in range(CTX):
            for q in range(NQ):
                sim4[q] = sim4[q] + (wc[cc, pl.ds(q * 16, 16)] *
                                     wc[cc, pl.ds(EMBED + q * 16, 16)])
        for q in range(NQ):
            simloc[pl.ds(e * EMBED + q * 16, 16)] = sim4[q]

        su4 = [zero] * NQ
        for j in range(NSAMP):
            for q in range(NQ):
                su4[q] = su4[q] + wn[j, pl.ds(q * 16, 16)]
        neg4 = [zero] * NQ
        for j in range(NSAMP):
            w = su4[j // 16][j % 16]
            for q in range(NQ):
                neg4[q] = neg4[q] + w * wn[j, pl.ds(EMBED + q * 16, 16)]
        for q in range(NQ):
            negloc[pl.ds(e * EMBED + q * 16, 16)] = neg4[q]

    for b in range(NBUF):
        issue(b, b)

    @pl.loop(0, per, step=NBUF)
    def _pipe(e):
        for b in range(NBUF):
            wait(b)
            compute(b, e + b)
            issue(b, jnp.minimum(e + b + NBUF, per - 1))

    for b in range(NBUF):
        wait(b)

    pltpu.sync_copy(simloc, sim_hbm.at[pl.ds(base * EMBED, per * EMBED)])
    pltpu.sync_copy(negloc, neg_hbm.at[pl.ds(base * EMBED, per * EMBED)])


def _loss_body(sim_ref, neg_ref, out_ref, *, batch):
    x = sim_ref[...]
    y = -neg_ref[...]

    def log_sigmoid(t):
        return jnp.minimum(t, 0.0) - jnp.log1p(jnp.exp(-jnp.abs(t)))

    total = jnp.sum(log_sigmoid(x)) + jnp.sum(log_sigmoid(y))
    out_ref[0, 0] = -total / float(batch)


def kernel(X, N, neg_samples, batch_size, embed_u, embed_v):
    del N, batch_size
    B = X.shape[0]
    nw = 32
    per = B // nw

    W = _combine_tables(embed_u.T, embed_v.T)

    XP = jnp.pad(X, ((0, 0), (0, CTXP - CTX))).reshape(-1)
    NP = neg_samples.reshape(-1)

    mesh = plsc.VectorSubcoreMesh(core_axis_name="c", subcore_axis_name="s")
    sc = pl.kernel(
        functools.partial(_sc_gather_body, per=per),
        out_type=(
            jax.ShapeDtypeStruct((B * EMBED,), jnp.float32),
            jax.ShapeDtypeStruct((B * EMBED,), jnp.float32),
        ),
        mesh=mesh,
        scratch_types=(
            pltpu.VMEM((per * CTXP,), jnp.int32),
            pltpu.VMEM((per * NSAMP,), jnp.int32),
            pltpu.VMEM((NBUF, CTX, 2 * EMBED), jnp.float32),
            pltpu.VMEM((NBUF, NSAMP, 2 * EMBED), jnp.float32),
            pltpu.VMEM((per * EMBED,), jnp.float32),
            pltpu.VMEM((per * EMBED,), jnp.float32),
            pltpu.SemaphoreType.DMA((NBUF,)),
        ),
    )
    sim, neg = sc(XP, NP, W)

    loss = pl.pallas_call(
        functools.partial(_loss_body, batch=B),
        out_shape=jax.ShapeDtypeStruct((1, 1), jnp.float32),
        out_specs=pl.BlockSpec(memory_space=pltpu.SMEM),
    )(sim.reshape(B * EMBED // 128, 128), neg.reshape(B * EMBED // 128, 128))
    return loss[0, 0]
